# baseline (speedup 1.0000x reference)
import functools

import jax
import jax.numpy as jnp
from jax import lax
from jax.experimental import pallas as pl
from jax.experimental.pallas import tpu as pltpu
from jax.experimental.pallas import tpu_sc as plsc

NC = 2
NS = 16
NW = NC * NS
LANES = 16

_MESH = dict(core_axis_name="c", subcore_axis_name="s",
             num_cores=NC, num_subcores=NS)
_TILED = pltpu.CompilerParams(use_tc_tiling_on_sc=True,
                              needs_layout_passes=False)


def _make_sc_format(V, D):
    ncol_full = V // 128
    vrem = V - ncol_full * 128
    nt = -(-ncol_full // NW)
    NT = nt + (nt % 2)
    nvec = D // LANES

    @functools.partial(
        pl.kernel,
        out_type=jax.ShapeDtypeStruct((V // 2, 2 * D), jnp.float32),
        mesh=plsc.VectorSubcoreMesh(**_MESH),
        compiler_params=_TILED,
        scratch_types=[
            pltpu.VMEM((2, D, 128), jnp.float32),
            pltpu.VMEM((2, 64, 2 * D), jnp.float32),
            pltpu.SemaphoreType.DMA,
            pltpu.SemaphoreType.DMA,
            pltpu.SemaphoreType.DMA,
            pltpu.SemaphoreType.DMA,
        ],
    )
    def sc_format(tbl_t, tail, pair, inbuf, outbuf, si0, si1, so0, so1):
        wid = lax.axis_index("s") * NC + lax.axis_index("c")
        sis = (si0, si1)
        sos = (so0, so1)

        def ceff(t):
            c = wid + NW * t
            return jnp.where(c < ncol_full, c, wid)

        def in_copy(t, b):
            c = ceff(t)
            return pltpu.make_async_copy(
                tbl_t.at[:, pl.ds(c * 128, 128)], inbuf.at[b], sis[b])

        def out_copy(t, b):
            c = ceff(t)
            return pltpu.make_async_copy(
                outbuf.at[b], pair.at[pl.ds(c * 64, 64)], sos[b])

        def transpose(b):
            def rbody(r, carry):
                for h in range(2):
                    col = jnp.full((LANES,), 2 * r + h, jnp.int32)
                    for k in range(nvec):
                        rows = lax.iota(jnp.int32, LANES) + (k * LANES)
                        v = plsc.load_gather(
                            inbuf, [jnp.full((LANES,), b, jnp.int32),
                                    rows, col])
                        outbuf[b, r, pl.ds(h * D + k * LANES, LANES)] = v
                return carry
            lax.fori_loop(0, 64, rbody, 0, unroll=4)

        in_copy(0, 0).start()
        in_copy(1, 1).start()

        def tbody(tt, carry):
            for b in range(2):
                t = 2 * tt + b
                in_copy(t, b).wait()

                @pl.when(tt >= 1)
                def _():
                    out_copy(t - 2, b).wait()

                transpose(b)
                out_copy(t, b).start()

                @pl.when(tt < NT // 2 - 1)
                def _():
                    in_copy(t + 2, b).start()
            return carry

        lax.fori_loop(0, NT // 2, tbody, 0)
        out_copy(NT - 2, 0).wait()
        out_copy(NT - 1, 1).wait()

        if vrem:
            @pl.when(wid == NW - 1)
            def _():
                pltpu.sync_copy(tail, outbuf.at[0, pl.ds(0, vrem // 2)])
                pltpu.sync_copy(
                    outbuf.at[0, pl.ds(0, vrem // 2)],
                    pair.at[pl.ds(ncol_full * 64, vrem // 2)])

    return sc_format


def _make_sc_pool(B, L, D):
    rows_w = B // NW
    CA = 104
    CB = L - CA
    nvec = D // LANES
    scale = 1.0 / L
    nidx = rows_w * L

    @functools.partial(
        pl.kernel,
        out_type=jax.ShapeDtypeStruct((B, D), jnp.float32),
        mesh=plsc.VectorSubcoreMesh(**_MESH),
        compiler_params=_TILED,
        scratch_types=[
            pltpu.VMEM((nidx,), jnp.int32),
            pltpu.VMEM((nidx + LANES,), jnp.int32),
            pltpu.VMEM((2, L, 2 * D), jnp.float32),
            pltpu.VMEM((rows_w, D), jnp.float32),
            pltpu.SemaphoreType.DMA,
            pltpu.SemaphoreType.DMA,
        ],
    )
    def sc_pool(pair_hbm, idx_hbm, out_hbm, idx_v, cb_v, buf, pooled_v,
                sem0, sem1):
        wid = lax.axis_index("s") * NC + lax.axis_index("c")
        pltpu.sync_copy(idx_hbm.at[pl.ds(wid * nidx, nidx)], idx_v)
        sems = (sem0, sem1)

        def pbody(i, carry):
            x = idx_v[pl.ds(i * LANES, LANES)]
            cb_v[pl.ds(i * LANES, LANES)] = (x & 1) * D
            idx_v[pl.ds(i * LANES, LANES)] = lax.shift_right_logical(x, 1)
            return carry
        lax.fori_loop(0, nidx // LANES, pbody, 0, unroll=8)

        def row_copies(r, b):
            o = r * L
            ca = pltpu.make_async_copy(
                pair_hbm.at[idx_v.at[pl.ds(o, CA)]],
                buf.at[b, pl.ds(0, CA)], sems[b])
            cb = pltpu.make_async_copy(
                pair_hbm.at[idx_v.at[pl.ds(o + CA, CB)]],
                buf.at[b, pl.ds(CA, CB)], sems[b])
            return ca, cb

        def issue(r, b):
            ca, cb = row_copies(r, b)
            ca.start()
            cb.start()

        def wait_row(r, b):
            ca, cb = row_copies(r, b)
            ca.wait()
            cb.wait()

        def acc_row(r, b):
            o = r * L
            bvec = jnp.full((LANES,), b, jnp.int32)
            ilane = lax.iota(jnp.int32, LANES)

            def jbody(jj, accs):
                cbv = cb_v[pl.ds(o + jj * 8, LANES)]
                for l in range(8):
                    j = jj * 8 + l
                    base = cbv[l] + ilane
                    jvec = jnp.full((LANES,), j, jnp.int32)
                    accs = tuple(
                        accs[k] + plsc.load_gather(
                            buf, [bvec, jvec, base + (k * LANES)])
                        for k in range(nvec))
                return accs
            z = jnp.zeros((LANES,), jnp.float32)
            accs = lax.fori_loop(0, L // 8, jbody, (z,) * nvec, unroll=2)
            for k in range(nvec):
                pooled_v[r, pl.ds(k * LANES, LANES)] = accs[k] * scale

        issue(0, 0)
        issue(1, 1)

        def obody(rr, carry):
            for b in range(2):
                r = 2 * rr + b
                wait_row(r, b)

                @pl.when(r + 2 < rows_w)
                def _():
                    issue(r + 2, b)

                acc_row(r, b)
            return carry

        lax.fori_loop(0, rows_w // 2, obody, 0)
        pltpu.sync_copy(pooled_v, out_hbm.at[pl.ds(wid * rows_w, rows_w)])

    return sc_pool


def _dense_body(pooled_ref, w1_ref, b1_ref, w2_ref, b2_ref, out_ref):
    p = pooled_ref[...]
    h = jnp.dot(p, w1_ref[...], preferred_element_type=jnp.float32)
    h = jnp.maximum(h + b1_ref[...], 0.0)
    logits = jnp.dot(h, w2_ref[...], preferred_element_type=jnp.float32)
    logits = logits + b2_ref[...]
    m = jnp.max(logits, axis=-1, keepdims=True)
    e = jnp.exp(logits - m)
    out_ref[...] = e / jnp.sum(e, axis=-1, keepdims=True)


def kernel(inputs, emb_table, W1, b1, W2, b2):
    B, L = inputs.shape
    V, D = emb_table.shape
    H = W1.shape[1]
    C = W2.shape[1]
    CP = 128

    idx_flat = inputs.astype(jnp.int32).reshape(-1)
    vrem = V - (V // 128) * 128
    tail = emb_table[V - vrem:].reshape(vrem // 2, 2 * D)
    pair = _make_sc_format(V, D)(emb_table.T, tail)
    pooled = _make_sc_pool(B, L, D)(pair, idx_flat)

    w2p = jnp.zeros((H, CP), jnp.float32).at[:, :C].set(W2)
    b2p = jnp.full((1, CP), -1e30, jnp.float32).at[0, :C].set(b2)
    b1r = b1.reshape(1, H)

    out = pl.pallas_call(
        _dense_body,
        out_shape=jax.ShapeDtypeStruct((B, CP), jnp.float32),
    )(pooled, W1, b1r, w2p, b2p)
    return out[:, :C]


# speedup vs baseline: 1.1205x; 1.1205x over previous
import functools

import jax
import jax.numpy as jnp
from jax import lax
from jax.experimental import pallas as pl
from jax.experimental.pallas import tpu as pltpu
from jax.experimental.pallas import tpu_sc as plsc

NC = 2
NS = 16
NW = NC * NS
LANES = 16

_MESH = dict(core_axis_name="c", subcore_axis_name="s",
             num_cores=NC, num_subcores=NS)
_TILED = pltpu.CompilerParams(use_tc_tiling_on_sc=True,
                              needs_layout_passes=False)


def _make_sc_format(V, D):
    ncol_full = V // 128
    vrem = V - ncol_full * 128
    nt = -(-ncol_full // NW)
    NT = nt + (nt % 2)
    nvec = D // LANES

    @functools.partial(
        pl.kernel,
        out_type=jax.ShapeDtypeStruct((V // 2, 2 * D), jnp.float32),
        mesh=plsc.VectorSubcoreMesh(**_MESH),
        compiler_params=_TILED,
        scratch_types=[
            pltpu.VMEM((2, D, 129), jnp.float32),
            pltpu.VMEM((2, 64, 2 * D), jnp.float32),
            pltpu.SemaphoreType.DMA,
            pltpu.SemaphoreType.DMA,
            pltpu.SemaphoreType.DMA,
            pltpu.SemaphoreType.DMA,
        ],
    )
    def sc_format(tbl_t, tail, pair, inbuf, outbuf, si0, si1, so0, so1):
        wid = lax.axis_index("s") * NC + lax.axis_index("c")
        sis = (si0, si1)
        sos = (so0, so1)

        def ceff(t):
            c = wid + NW * t
            return jnp.where(c < ncol_full, c, wid)

        def in_copy(t, b):
            c = ceff(t)
            return pltpu.make_async_copy(
                tbl_t.at[:, pl.ds(c * 128, 128)],
                inbuf.at[b, :, pl.ds(0, 128)], sis[b])

        def out_copy(t, b):
            c = ceff(t)
            return pltpu.make_async_copy(
                outbuf.at[b], pair.at[pl.ds(c * 64, 64)], sos[b])

        def transpose(b):
            def rbody(r, carry):
                for h in range(2):
                    col = jnp.full((LANES,), 2 * r + h, jnp.int32)
                    for k in range(nvec):
                        rows = lax.iota(jnp.int32, LANES) + (k * LANES)
                        v = plsc.load_gather(
                            inbuf, [jnp.full((LANES,), b, jnp.int32),
                                    rows, col])
                        outbuf[b, r, pl.ds(h * D + k * LANES, LANES)] = v
                return carry
            lax.fori_loop(0, 64, rbody, 0, unroll=4)

        in_copy(0, 0).start()
        in_copy(1, 1).start()

        def tbody(tt, carry):
            for b in range(2):
                t = 2 * tt + b
                in_copy(t, b).wait()

                @pl.when(tt >= 1)
                def _():
                    out_copy(t - 2, b).wait()

                transpose(b)
                out_copy(t, b).start()

                @pl.when(tt < NT // 2 - 1)
                def _():
                    in_copy(t + 2, b).start()
            return carry

        lax.fori_loop(0, NT // 2, tbody, 0)
        out_copy(NT - 2, 0).wait()
        out_copy(NT - 1, 1).wait()

        if vrem:
            @pl.when(wid == NW - 1)
            def _():
                pltpu.sync_copy(tail, outbuf.at[0, pl.ds(0, vrem // 2)])
                pltpu.sync_copy(
                    outbuf.at[0, pl.ds(0, vrem // 2)],
                    pair.at[pl.ds(ncol_full * 64, vrem // 2)])

    return sc_format


def _make_sc_pool(B, L, D):
    rows_w = B // NW
    CA = 104
    CB = L - CA
    nvec = D // LANES
    scale = 1.0 / L
    nidx = rows_w * L

    @functools.partial(
        pl.kernel,
        out_type=jax.ShapeDtypeStruct((B, D), jnp.float32),
        mesh=plsc.VectorSubcoreMesh(**_MESH),
        compiler_params=pltpu.CompilerParams(use_tc_tiling_on_sc=False),
        scratch_types=[
            pltpu.VMEM((nidx,), jnp.int32),
            pltpu.VMEM((2, L, D), jnp.float32),
            pltpu.VMEM((rows_w, D), jnp.float32),
            pltpu.SemaphoreType.DMA,
            pltpu.SemaphoreType.DMA,
        ],
    )
    def sc_pool(lin_hbm, idx_hbm, out_hbm, idx_v, buf, pooled_v, sem0, sem1):
        wid = lax.axis_index("s") * NC + lax.axis_index("c")
        pltpu.sync_copy(idx_hbm.at[pl.ds(wid * nidx, nidx)], idx_v)
        sems = (sem0, sem1)

        def row_copies(r, b):
            o = r * L
            ca = pltpu.make_async_copy(
                lin_hbm.at[idx_v.at[pl.ds(o, CA)]],
                buf.at[b, pl.ds(0, CA)], sems[b])
            cb = pltpu.make_async_copy(
                lin_hbm.at[idx_v.at[pl.ds(o + CA, CB)]],
                buf.at[b, pl.ds(CA, CB)], sems[b])
            return ca, cb

        def issue(r, b):
            ca, cb = row_copies(r, b)
            ca.start()
            cb.start()

        def wait_row(r, b):
            ca, cb = row_copies(r, b)
            ca.wait()
            cb.wait()

        def acc_row(r, b):
            def jbody(j, accs):
                return tuple(
                    accs[k] + buf[b, j, pl.ds(k * LANES, LANES)]
                    for k in range(nvec))
            z = jnp.zeros((LANES,), jnp.float32)
            accs = lax.fori_loop(0, L, jbody, (z,) * nvec, unroll=8)
            for k in range(nvec):
                pooled_v[r, pl.ds(k * LANES, LANES)] = accs[k] * scale

        issue(0, 0)
        issue(1, 1)

        def obody(rr, carry):
            for b in range(2):
                r = 2 * rr + b
                wait_row(r, b)

                @pl.when(r + 2 < rows_w)
                def _():
                    issue(r + 2, b)

                acc_row(r, b)
            return carry

        lax.fori_loop(0, rows_w // 2, obody, 0)
        pltpu.sync_copy(pooled_v, out_hbm.at[pl.ds(wid * rows_w, rows_w)])

    return sc_pool


def _dense_body(pooled_ref, w1_ref, b1_ref, w2_ref, b2_ref, out_ref):
    p = pooled_ref[...]
    h = jnp.dot(p, w1_ref[...], preferred_element_type=jnp.float32)
    h = jnp.maximum(h + b1_ref[...], 0.0)
    logits = jnp.dot(h, w2_ref[...], preferred_element_type=jnp.float32)
    logits = logits + b2_ref[...]
    m = jnp.max(logits, axis=-1, keepdims=True)
    e = jnp.exp(logits - m)
    out_ref[...] = e / jnp.sum(e, axis=-1, keepdims=True)


def kernel(inputs, emb_table, W1, b1, W2, b2):
    B, L = inputs.shape
    V, D = emb_table.shape
    H = W1.shape[1]
    C = W2.shape[1]
    CP = 128

    idx_flat = inputs.astype(jnp.int32).reshape(-1)
    vrem = V - (V // 128) * 128
    tail = emb_table[V - vrem:].reshape(vrem // 2, 2 * D)
    pair = _make_sc_format(V, D)(emb_table.T, tail)
    lin = pair.reshape(V, D)
    pooled = _make_sc_pool(B, L, D)(lin, idx_flat)

    w2p = jnp.zeros((H, CP), jnp.float32).at[:, :C].set(W2)
    b2p = jnp.full((1, CP), -1e30, jnp.float32).at[0, :C].set(b2)
    b1r = b1.reshape(1, H)

    out = pl.pallas_call(
        _dense_body,
        out_shape=jax.ShapeDtypeStruct((B, CP), jnp.float32),
    )(pooled, W1, b1r, w2p, b2p)
    return out[:, :C]
